# baseline (speedup 1.0000x reference)
import jax
import jax.numpy as jnp
from jax import lax
from jax.experimental import pallas as pl
from jax.experimental.pallas import tpu as pltpu
from jax.experimental.pallas import tpu_sc as plsc

B = 4
S = 2048
DIM = 768
COND_SIZE = 6
COND_DIMS = 128

NC = 2
NS = 16
NW = NC * NS
SROWS = S // NW
CHUNK = 16
NCHUNK = SROWS // CHUNK
LANES = 16
NGROUPS = DIM // LANES


KSTEPS = 6
JBLK = DIM // KSTEPS


def _affine_body(cid_ref, cond_emb_ref, wh_ref, wg_ref, wb_ref, clg_ref,
                 clb_ref, gb_ref, conds_ref, gt_ref, bt_ref):
  k = pl.program_id(0)

  @pl.when(k == 0)
  def _():
    conds_ref[...] = lax.dot_general(cond_emb_ref[...], wh_ref[...],
                                     (((1,), (1,)), ((), ())),
                                     preferred_element_type=jnp.float32)

  gt_ref[pl.ds(k * JBLK, JBLK), :] = lax.dot_general(
      wg_ref[...], conds_ref[...], (((1,), (1,)), ((), ())),
      preferred_element_type=jnp.float32)
  bt_ref[pl.ds(k * JBLK, JBLK), :] = lax.dot_general(
      wb_ref[...], conds_ref[...], (((1,), (1,)), ((), ())),
      preferred_element_type=jnp.float32)

  @pl.when(k == KSTEPS - 1)
  def _():
    onehot = (cid_ref[...][:, None]
              == lax.broadcasted_iota(jnp.int32, (B, COND_SIZE), 1)
              ).astype(jnp.float32)
    gb_ref[0:B, :] = clg_ref[...][None, :] + lax.dot_general(
        onehot, gt_ref[...], (((1,), (1,)), ((), ())),
        preferred_element_type=jnp.float32)
    gb_ref[B:2 * B, :] = clb_ref[...][None, :] + lax.dot_general(
        onehot, bt_ref[...], (((1,), (1,)), ((), ())),
        preferred_element_type=jnp.float32)


def _affine_params(condition_ids, cond_emb, W_hidden, W_gma, W_bta, clngma,
                   clnbta):
  return pl.pallas_call(
      _affine_body,
      grid=(KSTEPS,),
      in_specs=[
          pl.BlockSpec((B,), lambda k: (0,)),
          pl.BlockSpec((COND_SIZE, COND_DIMS), lambda k: (0, 0)),
          pl.BlockSpec((DIM, COND_DIMS), lambda k: (0, 0)),
          pl.BlockSpec((JBLK, DIM), lambda k: (k, 0)),
          pl.BlockSpec((JBLK, DIM), lambda k: (k, 0)),
          pl.BlockSpec((DIM,), lambda k: (0,)),
          pl.BlockSpec((DIM,), lambda k: (0,)),
      ],
      out_specs=pl.BlockSpec((2 * B, DIM), lambda k: (0, 0)),
      out_shape=jax.ShapeDtypeStruct((2 * B, DIM), jnp.float32),
      scratch_shapes=[
          pltpu.VMEM((COND_SIZE, DIM), jnp.float32),
          pltpu.VMEM((DIM, COND_SIZE), jnp.float32),
          pltpu.VMEM((DIM, COND_SIZE), jnp.float32),
      ],
  )(condition_ids, cond_emb, W_hidden, W_gma, W_bta, clngma, clnbta)


def _sc_body(ids_hbm, word_hbm, pos_hbm, gb_hbm, out_hbm,
             w0_v, w1_v, pos_v, idx_v, gb_v,
             g0_sem, g1_sem, pp_sem, o0_sem, o1_sem, i_sem):
  wid = lax.axis_index("s") * NC + lax.axis_index("c")
  s0 = wid * SROWS

  word_bufs = (w0_v, w1_v)
  g_sems = (g0_sem, g1_sem)
  o_sems = (o0_sem, o1_sem)

  idx_descs = [
      pltpu.async_copy(ids_hbm.at[b, pl.ds(s0 + c * CHUNK, CHUNK)],
                       idx_v.at[c, pl.ds(b * CHUNK, CHUNK)], i_sem)
      for c in range(NCHUNK) for b in range(B)
  ]
  gb_desc = pltpu.async_copy(gb_hbm, gb_v, pp_sem)
  for d in idx_descs:
    d.wait()
  gb_desc.wait()

  def issue_gather(k):
    return pltpu.async_copy(word_hbm.at[idx_v.at[k]], word_bufs[k % 2],
                            g_sems[k % 2])

  def issue_pos(k):
    return pltpu.async_copy(pos_hbm.at[pl.ds(s0 + k * CHUNK, CHUNK)],
                            pos_v, pp_sem)

  gather_descs = {0: issue_gather(0)}
  pos_descs = {0: issue_pos(0)}
  out_descs = {}

  for k in range(NCHUNK):
    slot = k % 2
    if k + 1 < NCHUNK:
      if k - 1 >= 0:
        for d in out_descs[k - 1]:
          d.wait()
      gather_descs[k + 1] = issue_gather(k + 1)
    gather_descs[k].wait()
    pos_descs[k].wait()

    wbuf = word_bufs[slot]

    @plsc.parallel_loop(0, NGROUPS, step=1, unroll=1)
    def gbody(g, wbuf=wbuf):
      gs = pl.ds(lax.mul(g, LANES), LANES)
      gms = [gb_v[b, gs] for b in range(B)]
      bts = [gb_v[B + b, gs] for b in range(B)]
      for r in range(CHUNK):
        p16 = pos_v[r, gs]
        for b in range(B):
          row = b * CHUNK + r
          wbuf[row, gs] = (wbuf[row, gs] + p16) * gms[b] + bts[b]

    if k + 1 < NCHUNK:
      pos_descs[k + 1] = issue_pos(k + 1)

    out_descs[k] = tuple(
        pltpu.async_copy(wbuf.at[pl.ds(b * CHUNK, CHUNK)],
                         out_hbm.at[b, pl.ds(s0 + k * CHUNK, CHUNK)],
                         o_sems[slot])
        for b in range(B))

  for k in (NCHUNK - 2, NCHUNK - 1):
    for d in out_descs[k]:
      d.wait()


def _sc_gather_affine(ids, word_emb, pos_emb, gb):
  kern = pl.kernel(
      _sc_body,
      out_type=jax.ShapeDtypeStruct((B, S, DIM), jnp.float32),
      mesh=plsc.VectorSubcoreMesh(core_axis_name="c", subcore_axis_name="s",
                                  num_cores=NC, num_subcores=NS),
      scratch_types=[
          pltpu.VMEM((B * CHUNK, DIM), jnp.float32),
          pltpu.VMEM((B * CHUNK, DIM), jnp.float32),
          pltpu.VMEM((CHUNK, DIM), jnp.float32),
          pltpu.VMEM((NCHUNK, B * CHUNK), jnp.int32),
          pltpu.VMEM((2 * B, DIM), jnp.float32),
          pltpu.SemaphoreType.DMA,
          pltpu.SemaphoreType.DMA,
          pltpu.SemaphoreType.DMA,
          pltpu.SemaphoreType.DMA,
          pltpu.SemaphoreType.DMA,
          pltpu.SemaphoreType.DMA,
      ],
  )
  return kern(ids, word_emb, pos_emb, gb)


def kernel(input_ids, condition_ids, word_emb, pos_emb, cond_emb, W_hidden,
           W_gma, W_bta, clngma, clnbta):
  if input_ids.dtype != jnp.int32:
    input_ids = input_ids.astype(jnp.int32)
  if condition_ids.dtype != jnp.int32:
    condition_ids = condition_ids.astype(jnp.int32)
  gb = _affine_params(condition_ids, cond_emb, W_hidden, W_gma, W_bta,
                      clngma, clnbta)
  return _sc_gather_affine(input_ids, word_emb, pos_emb, gb)


# speedup vs baseline: 1.0338x; 1.0338x over previous
import jax
import jax.numpy as jnp
from jax import lax
from jax.experimental import pallas as pl
from jax.experimental.pallas import tpu as pltpu
from jax.experimental.pallas import tpu_sc as plsc

B = 4
S = 2048
DIM = 768
COND_SIZE = 6
COND_DIMS = 128

NC = 2
NS = 16
NW = NC * NS
SROWS = S // NW
CHUNK = 16
NCHUNK = SROWS // CHUNK
LANES = 16
NGROUPS = DIM // LANES


def _affine_body(cid_ref, cond_emb_ref, wh_ref, wg_ref, wb_ref, clg_ref,
                 clb_ref, gb_ref):
  conds6 = lax.dot_general(cond_emb_ref[...], wh_ref[...],
                           (((1,), (1,)), ((), ())),
                           preferred_element_type=jnp.float32)
  gamma6 = clg_ref[...][None, :] + lax.dot_general(
      conds6, wg_ref[...], (((1,), (1,)), ((), ())),
      preferred_element_type=jnp.float32)
  beta6 = clb_ref[...][None, :] + lax.dot_general(
      conds6, wb_ref[...], (((1,), (1,)), ((), ())),
      preferred_element_type=jnp.float32)
  onehot = (cid_ref[...][:, None]
            == lax.broadcasted_iota(jnp.int32, (B, COND_SIZE), 1)
            ).astype(jnp.float32)
  gb_ref[0:B, :] = lax.dot_general(onehot, gamma6, (((1,), (0,)), ((), ())),
                                   preferred_element_type=jnp.float32)
  gb_ref[B:2 * B, :] = lax.dot_general(onehot, beta6,
                                       (((1,), (0,)), ((), ())),
                                       preferred_element_type=jnp.float32)


def _affine_params(condition_ids, cond_emb, W_hidden, W_gma, W_bta, clngma,
                   clnbta):
  return pl.pallas_call(
      _affine_body,
      out_shape=jax.ShapeDtypeStruct((2 * B, DIM), jnp.float32),
  )(condition_ids, cond_emb, W_hidden, W_gma, W_bta, clngma, clnbta)


def _sc_body(ids_hbm, word_hbm, pos_hbm, gb_hbm, out_hbm,
             w0_v, w1_v, pos_v, idx_v, gb_v,
             g0_sem, g1_sem, pp_sem, o0_sem, o1_sem, i_sem):
  wid = lax.axis_index("s") * NC + lax.axis_index("c")
  s0 = wid * SROWS

  word_bufs = (w0_v, w1_v)
  g_sems = (g0_sem, g1_sem)
  o_sems = (o0_sem, o1_sem)

  idx_descs = [
      pltpu.async_copy(ids_hbm.at[b, pl.ds(s0 + c * CHUNK, CHUNK)],
                       idx_v.at[c, pl.ds(b * CHUNK, CHUNK)], i_sem)
      for c in range(NCHUNK) for b in range(B)
  ]
  gb_desc = pltpu.async_copy(gb_hbm, gb_v, pp_sem)
  for d in idx_descs:
    d.wait()
  gb_desc.wait()

  def issue_gather(k):
    return pltpu.async_copy(word_hbm.at[idx_v.at[k]], word_bufs[k % 2],
                            g_sems[k % 2])

  def issue_pos(k):
    return pltpu.async_copy(pos_hbm.at[pl.ds(s0 + k * CHUNK, CHUNK)],
                            pos_v, pp_sem)

  gather_descs = {0: issue_gather(0)}
  pos_descs = {0: issue_pos(0)}
  out_descs = {}

  for k in range(NCHUNK):
    slot = k % 2
    if k + 1 < NCHUNK:
      if k - 1 >= 0:
        for d in out_descs[k - 1]:
          d.wait()
      gather_descs[k + 1] = issue_gather(k + 1)
    gather_descs[k].wait()
    pos_descs[k].wait()

    wbuf = word_bufs[slot]

    @plsc.parallel_loop(0, NGROUPS, step=1, unroll=1)
    def gbody(g, wbuf=wbuf):
      gs = pl.ds(lax.mul(g, LANES), LANES)
      gms = [gb_v[b, gs] for b in range(B)]
      bts = [gb_v[B + b, gs] for b in range(B)]
      for r in range(CHUNK):
        p16 = pos_v[r, gs]
        for b in range(B):
          row = b * CHUNK + r
          wbuf[row, gs] = (wbuf[row, gs] + p16) * gms[b] + bts[b]

    if k + 1 < NCHUNK:
      pos_descs[k + 1] = issue_pos(k + 1)

    out_descs[k] = tuple(
        pltpu.async_copy(wbuf.at[pl.ds(b * CHUNK, CHUNK)],
                         out_hbm.at[b, pl.ds(s0 + k * CHUNK, CHUNK)],
                         o_sems[slot])
        for b in range(B))

  for k in (NCHUNK - 2, NCHUNK - 1):
    for d in out_descs[k]:
      d.wait()


def _sc_gather_affine(ids, word_emb, pos_emb, gb):
  kern = pl.kernel(
      _sc_body,
      out_type=jax.ShapeDtypeStruct((B, S, DIM), jnp.float32),
      mesh=plsc.VectorSubcoreMesh(core_axis_name="c", subcore_axis_name="s",
                                  num_cores=NC, num_subcores=NS),
      scratch_types=[
          pltpu.VMEM((B * CHUNK, DIM), jnp.float32),
          pltpu.VMEM((B * CHUNK, DIM), jnp.float32),
          pltpu.VMEM((CHUNK, DIM), jnp.float32),
          pltpu.VMEM((NCHUNK, B * CHUNK), jnp.int32),
          pltpu.VMEM((2 * B, DIM), jnp.float32),
          pltpu.SemaphoreType.DMA,
          pltpu.SemaphoreType.DMA,
          pltpu.SemaphoreType.DMA,
          pltpu.SemaphoreType.DMA,
          pltpu.SemaphoreType.DMA,
          pltpu.SemaphoreType.DMA,
      ],
  )
  return kern(ids, word_emb, pos_emb, gb)


def kernel(input_ids, condition_ids, word_emb, pos_emb, cond_emb, W_hidden,
           W_gma, W_bta, clngma, clnbta):
  if input_ids.dtype != jnp.int32:
    input_ids = input_ids.astype(jnp.int32)
  if condition_ids.dtype != jnp.int32:
    condition_ids = condition_ids.astype(jnp.int32)
  gb = _affine_params(condition_ids, cond_emb, W_hidden, W_gma, W_bta,
                      clngma, clnbta)
  return _sc_gather_affine(input_ids, word_emb, pos_emb, gb)


# speedup vs baseline: 1.0536x; 1.0192x over previous
import jax
import jax.numpy as jnp
from jax import lax
from jax.experimental import pallas as pl
from jax.experimental.pallas import tpu as pltpu
from jax.experimental.pallas import tpu_sc as plsc

B = 4
S = 2048
DIM = 768
COND_SIZE = 6
COND_DIMS = 128

NC = 2
NS = 16
NW = NC * NS
SROWS = S // NW
CHUNK = 8
NCHUNK = SROWS // CHUNK
NBUF = 4
LANES = 16
NGROUPS = DIM // LANES


def _affine_body(cid_ref, cond_emb_ref, wh_ref, wg_ref, wb_ref, clg_ref,
                 clb_ref, gb_ref):
  conds6 = lax.dot_general(cond_emb_ref[...], wh_ref[...],
                           (((1,), (1,)), ((), ())),
                           preferred_element_type=jnp.float32)
  gamma6 = clg_ref[...][None, :] + lax.dot_general(
      conds6, wg_ref[...], (((1,), (1,)), ((), ())),
      preferred_element_type=jnp.float32)
  beta6 = clb_ref[...][None, :] + lax.dot_general(
      conds6, wb_ref[...], (((1,), (1,)), ((), ())),
      preferred_element_type=jnp.float32)
  onehot = (cid_ref[...][:, None]
            == lax.broadcasted_iota(jnp.int32, (B, COND_SIZE), 1)
            ).astype(jnp.float32)
  gb_ref[0:B, :] = lax.dot_general(onehot, gamma6, (((1,), (0,)), ((), ())),
                                   preferred_element_type=jnp.float32)
  gb_ref[B:2 * B, :] = lax.dot_general(onehot, beta6,
                                       (((1,), (0,)), ((), ())),
                                       preferred_element_type=jnp.float32)


def _affine_params(condition_ids, cond_emb, W_hidden, W_gma, W_bta, clngma,
                   clnbta):
  return pl.pallas_call(
      _affine_body,
      out_shape=jax.ShapeDtypeStruct((2 * B, DIM), jnp.float32),
  )(condition_ids, cond_emb, W_hidden, W_gma, W_bta, clngma, clnbta)


def _sc_body(ids_hbm, word_hbm, pos_hbm, gb_hbm, out_hbm,
             w0_v, w1_v, w2_v, w3_v, p0_v, p1_v, idx_v, gb_v,
             g0_sem, g1_sem, g2_sem, g3_sem, pp0_sem, pp1_sem, o0_sem,
             o1_sem, i_sem):
  wid = lax.axis_index("s") * NC + lax.axis_index("c")
  s0 = wid * SROWS

  word_bufs = (w0_v, w1_v, w2_v, w3_v)
  pos_bufs = (p0_v, p1_v)
  g_sems = (g0_sem, g1_sem, g2_sem, g3_sem)
  o_sems = (o0_sem, o1_sem)

  idx_descs = [
      pltpu.async_copy(ids_hbm.at[b, pl.ds(s0 + c * CHUNK, CHUNK)],
                       idx_v.at[c, pl.ds(b * CHUNK, CHUNK)], i_sem)
      for c in range(NCHUNK) for b in range(B)
  ]
  gb_desc = pltpu.async_copy(gb_hbm, gb_v, i_sem)
  for d in idx_descs:
    d.wait()
  gb_desc.wait()

  def issue_gather(k):
    return pltpu.async_copy(word_hbm.at[idx_v.at[k]], word_bufs[k % NBUF],
                            g_sems[k % NBUF])

  def issue_pos(k):
    return pltpu.async_copy(pos_hbm.at[pl.ds(s0 + k * CHUNK, CHUNK)],
                            pos_bufs[k % 2], (pp0_sem, pp1_sem)[k % 2])

  gather_descs = {k: issue_gather(k) for k in range(NBUF - 1)}
  pos_descs = {k: issue_pos(k) for k in range(2)}
  out_descs = {}

  for k in range(NCHUNK):
    if k + NBUF - 1 < NCHUNK:
      if k - 1 >= 0:
        for d in out_descs[k - 1]:
          d.wait()
      gather_descs[k + NBUF - 1] = issue_gather(k + NBUF - 1)
    gather_descs[k].wait()
    pos_descs[k].wait()

    wbuf = word_bufs[k % NBUF]
    pbuf = pos_bufs[k % 2]

    @plsc.parallel_loop(0, NGROUPS, step=1, unroll=1)
    def gbody(g, wbuf=wbuf, pbuf=pbuf):
      gs = pl.ds(lax.mul(g, LANES), LANES)
      gms = [gb_v[b, gs] for b in range(B)]
      bts = [gb_v[B + b, gs] for b in range(B)]
      for r in range(CHUNK):
        p16 = pbuf[r, gs]
        for b in range(B):
          row = b * CHUNK + r
          wbuf[row, gs] = (wbuf[row, gs] + p16) * gms[b] + bts[b]

    if k + 2 < NCHUNK:
      pos_descs[k + 2] = issue_pos(k + 2)

    out_descs[k] = tuple(
        pltpu.async_copy(wbuf.at[pl.ds(b * CHUNK, CHUNK)],
                         out_hbm.at[b, pl.ds(s0 + k * CHUNK, CHUNK)],
                         o_sems[k % 2])
        for b in range(B))

  for k in range(max(0, NCHUNK - NBUF), NCHUNK):
    for d in out_descs[k]:
      d.wait()


def _sc_gather_affine(ids, word_emb, pos_emb, gb):
  kern = pl.kernel(
      _sc_body,
      out_type=jax.ShapeDtypeStruct((B, S, DIM), jnp.float32),
      mesh=plsc.VectorSubcoreMesh(core_axis_name="c", subcore_axis_name="s",
                                  num_cores=NC, num_subcores=NS),
      scratch_types=[
          pltpu.VMEM((B * CHUNK, DIM), jnp.float32),
          pltpu.VMEM((B * CHUNK, DIM), jnp.float32),
          pltpu.VMEM((B * CHUNK, DIM), jnp.float32),
          pltpu.VMEM((B * CHUNK, DIM), jnp.float32),
          pltpu.VMEM((CHUNK, DIM), jnp.float32),
          pltpu.VMEM((CHUNK, DIM), jnp.float32),
          pltpu.VMEM((NCHUNK, B * CHUNK), jnp.int32),
          pltpu.VMEM((2 * B, DIM), jnp.float32),
          pltpu.SemaphoreType.DMA,
          pltpu.SemaphoreType.DMA,
          pltpu.SemaphoreType.DMA,
          pltpu.SemaphoreType.DMA,
          pltpu.SemaphoreType.DMA,
          pltpu.SemaphoreType.DMA,
          pltpu.SemaphoreType.DMA,
          pltpu.SemaphoreType.DMA,
          pltpu.SemaphoreType.DMA,
      ],
  )
  return kern(ids, word_emb, pos_emb, gb)


def kernel(input_ids, condition_ids, word_emb, pos_emb, cond_emb, W_hidden,
           W_gma, W_bta, clngma, clnbta):
  if input_ids.dtype != jnp.int32:
    input_ids = input_ids.astype(jnp.int32)
  if condition_ids.dtype != jnp.int32:
    condition_ids = condition_ids.astype(jnp.int32)
  gb = _affine_params(condition_ids, cond_emb, W_hidden, W_gma, W_bta,
                      clngma, clnbta)
  return _sc_gather_affine(input_ids, word_emb, pos_emb, gb)


# speedup vs baseline: 1.0688x; 1.0144x over previous
import jax
import jax.numpy as jnp
from jax import lax
from jax.experimental import pallas as pl
from jax.experimental.pallas import tpu as pltpu
from jax.experimental.pallas import tpu_sc as plsc

B = 4
S = 2048
DIM = 768
COND_SIZE = 6
COND_DIMS = 128

NC = 2
NS = 16
NW = NC * NS
SROWS = S // NW
CHUNK = 8
NCHUNK = SROWS // CHUNK
NBUF = 4
LANES = 16
NGROUPS = DIM // LANES


def _affine_body(cid_ref, cond_emb_ref, wh_ref, wg_ref, wb_ref, clg_ref,
                 clb_ref, gb_ref):
  conds6 = lax.dot_general(cond_emb_ref[...], wh_ref[...],
                           (((1,), (1,)), ((), ())),
                           preferred_element_type=jnp.float32)
  gamma6 = clg_ref[...][None, :] + lax.dot_general(
      conds6, wg_ref[...], (((1,), (1,)), ((), ())),
      preferred_element_type=jnp.float32)
  beta6 = clb_ref[...][None, :] + lax.dot_general(
      conds6, wb_ref[...], (((1,), (1,)), ((), ())),
      preferred_element_type=jnp.float32)
  onehot = (cid_ref[...][:, None]
            == lax.broadcasted_iota(jnp.int32, (B, COND_SIZE), 1)
            ).astype(jnp.float32)
  gb_ref[0:B, :] = lax.dot_general(onehot, gamma6, (((1,), (0,)), ((), ())),
                                   preferred_element_type=jnp.float32)
  gb_ref[B:2 * B, :] = lax.dot_general(onehot, beta6,
                                       (((1,), (0,)), ((), ())),
                                       preferred_element_type=jnp.float32)


def _affine_params(condition_ids, cond_emb, W_hidden, W_gma, W_bta, clngma,
                   clnbta):
  return pl.pallas_call(
      _affine_body,
      out_shape=jax.ShapeDtypeStruct((2 * B, DIM), jnp.float32),
  )(condition_ids, cond_emb, W_hidden, W_gma, W_bta, clngma, clnbta)


def _sc_body(ids_hbm, word_hbm, pos_hbm, gb_hbm, out_hbm,
             w0_v, w1_v, w2_v, w3_v, p0_v, p1_v, idx_v, gb_v,
             g0_sem, g1_sem, g2_sem, g3_sem, pp0_sem, pp1_sem, o0_sem,
             o1_sem, i_sem):
  wid = lax.axis_index("s") * NC + lax.axis_index("c")
  s0 = wid * SROWS

  word_bufs = (w0_v, w1_v, w2_v, w3_v)
  pos_bufs = (p0_v, p1_v)
  g_sems = (g0_sem, g1_sem, g2_sem, g3_sem)
  o_sems = (o0_sem, o1_sem)

  idx_descs = [
      pltpu.async_copy(ids_hbm.at[b, pl.ds(s0 + c * CHUNK, CHUNK)],
                       idx_v.at[c, pl.ds(b * CHUNK, CHUNK)], i_sem)
      for c in range(NCHUNK) for b in range(B)
  ]
  gb_desc = pltpu.async_copy(gb_hbm, gb_v, i_sem)

  def issue_gather(k):
    return pltpu.async_copy(word_hbm.at[idx_v.at[k]], word_bufs[k % NBUF],
                            g_sems[k % NBUF])

  def issue_pos(k):
    return pltpu.async_copy(pos_hbm.at[pl.ds(s0 + k * CHUNK, CHUNK)],
                            pos_bufs[k % 2], (pp0_sem, pp1_sem)[k % 2])

  pos_descs = {k: issue_pos(k) for k in range(2)}
  gather_descs = {}
  for k in range(NBUF - 1):
    for d in idx_descs[k * B:(k + 1) * B]:
      d.wait()
    gather_descs[k] = issue_gather(k)
  for d in idx_descs[(NBUF - 1) * B:]:
    d.wait()
  gb_desc.wait()
  out_descs = {}

  for k in range(NCHUNK):
    if k + NBUF - 1 < NCHUNK:
      if k - 1 >= 0:
        for d in out_descs[k - 1]:
          d.wait()
      gather_descs[k + NBUF - 1] = issue_gather(k + NBUF - 1)
    gather_descs[k].wait()
    pos_descs[k].wait()

    wbuf = word_bufs[k % NBUF]
    pbuf = pos_bufs[k % 2]

    @plsc.parallel_loop(0, NGROUPS, step=1, unroll=1)
    def gbody(g, wbuf=wbuf, pbuf=pbuf):
      gs = pl.ds(lax.mul(g, LANES), LANES)
      gms = [gb_v[b, gs] for b in range(B)]
      bts = [gb_v[B + b, gs] for b in range(B)]
      for r in range(CHUNK):
        p16 = pbuf[r, gs]
        for b in range(B):
          row = b * CHUNK + r
          wbuf[row, gs] = (wbuf[row, gs] + p16) * gms[b] + bts[b]

    if k + 2 < NCHUNK:
      pos_descs[k + 2] = issue_pos(k + 2)

    out_descs[k] = tuple(
        pltpu.async_copy(wbuf.at[pl.ds(b * CHUNK, CHUNK)],
                         out_hbm.at[b, pl.ds(s0 + k * CHUNK, CHUNK)],
                         o_sems[k % 2])
        for b in range(B))

  for k in range(max(0, NCHUNK - NBUF), NCHUNK):
    for d in out_descs[k]:
      d.wait()


def _sc_gather_affine(ids, word_emb, pos_emb, gb):
  kern = pl.kernel(
      _sc_body,
      out_type=jax.ShapeDtypeStruct((B, S, DIM), jnp.float32),
      mesh=plsc.VectorSubcoreMesh(core_axis_name="c", subcore_axis_name="s",
                                  num_cores=NC, num_subcores=NS),
      scratch_types=[
          pltpu.VMEM((B * CHUNK, DIM), jnp.float32),
          pltpu.VMEM((B * CHUNK, DIM), jnp.float32),
          pltpu.VMEM((B * CHUNK, DIM), jnp.float32),
          pltpu.VMEM((B * CHUNK, DIM), jnp.float32),
          pltpu.VMEM((CHUNK, DIM), jnp.float32),
          pltpu.VMEM((CHUNK, DIM), jnp.float32),
          pltpu.VMEM((NCHUNK, B * CHUNK), jnp.int32),
          pltpu.VMEM((2 * B, DIM), jnp.float32),
          pltpu.SemaphoreType.DMA,
          pltpu.SemaphoreType.DMA,
          pltpu.SemaphoreType.DMA,
          pltpu.SemaphoreType.DMA,
          pltpu.SemaphoreType.DMA,
          pltpu.SemaphoreType.DMA,
          pltpu.SemaphoreType.DMA,
          pltpu.SemaphoreType.DMA,
          pltpu.SemaphoreType.DMA,
      ],
  )
  return kern(ids, word_emb, pos_emb, gb)


def kernel(input_ids, condition_ids, word_emb, pos_emb, cond_emb, W_hidden,
           W_gma, W_bta, clngma, clnbta):
  if input_ids.dtype != jnp.int32:
    input_ids = input_ids.astype(jnp.int32)
  if condition_ids.dtype != jnp.int32:
    condition_ids = condition_ids.astype(jnp.int32)
  gb = _affine_params(condition_ids, cond_emb, W_hidden, W_gma, W_bta,
                      clngma, clnbta)
  return _sc_gather_affine(input_ids, word_emb, pos_emb, gb)


# speedup vs baseline: 1.0695x; 1.0007x over previous
import jax
import jax.numpy as jnp
from jax import lax
from jax.experimental import pallas as pl
from jax.experimental.pallas import tpu as pltpu
from jax.experimental.pallas import tpu_sc as plsc

B = 4
S = 2048
DIM = 768
COND_SIZE = 6
COND_DIMS = 128

NC = 2
NS = 16
NW = NC * NS
SROWS = S // NW
CHUNK = 8
NCHUNK = SROWS // CHUNK
NBUF = 4
LANES = 16
NGROUPS = DIM // LANES


def _affine_body(cid_ref, cond_emb_ref, wh_ref, wg_ref, wb_ref, clg_ref,
                 clb_ref, gb_ref):
  conds6 = lax.dot_general(cond_emb_ref[...], wh_ref[...],
                           (((1,), (1,)), ((), ())),
                           preferred_element_type=jnp.float32)
  gamma6 = clg_ref[...][None, :] + lax.dot_general(
      conds6, wg_ref[...], (((1,), (1,)), ((), ())),
      preferred_element_type=jnp.float32)
  beta6 = clb_ref[...][None, :] + lax.dot_general(
      conds6, wb_ref[...], (((1,), (1,)), ((), ())),
      preferred_element_type=jnp.float32)
  onehot = (cid_ref[...][:, None]
            == lax.broadcasted_iota(jnp.int32, (B, COND_SIZE), 1)
            ).astype(jnp.float32)
  gb_ref[0:B, :] = lax.dot_general(onehot, gamma6, (((1,), (0,)), ((), ())),
                                   preferred_element_type=jnp.float32)
  gb_ref[B:2 * B, :] = lax.dot_general(onehot, beta6,
                                       (((1,), (0,)), ((), ())),
                                       preferred_element_type=jnp.float32)


def _affine_params(condition_ids, cond_emb, W_hidden, W_gma, W_bta, clngma,
                   clnbta):
  return pl.pallas_call(
      _affine_body,
      out_shape=jax.ShapeDtypeStruct((2 * B, DIM), jnp.float32),
  )(condition_ids, cond_emb, W_hidden, W_gma, W_bta, clngma, clnbta)


def _sc_body(ids_hbm, word_hbm, pos_hbm, gb_hbm, out_hbm,
             w0_v, w1_v, w2_v, w3_v, p0_v, p1_v, idx_v, gb_v,
             g0_sem, g1_sem, g2_sem, g3_sem, pp0_sem, pp1_sem, o0_sem,
             o1_sem, i_sem):
  wid = lax.axis_index("s") * NC + lax.axis_index("c")
  s0 = wid * SROWS

  word_bufs = (w0_v, w1_v, w2_v, w3_v)
  pos_bufs = (p0_v, p1_v)
  g_sems = (g0_sem, g1_sem, g2_sem, g3_sem)
  o_sems = (o0_sem, o1_sem)

  idx_descs = [
      pltpu.async_copy(ids_hbm.at[b, pl.ds(s0 + c * CHUNK, CHUNK)],
                       idx_v.at[c, pl.ds(b * CHUNK, CHUNK)],
                       g_sems[c % NBUF] if c < NBUF - 1 else i_sem)
      for c in range(NCHUNK) for b in range(B)
  ]
  gb_desc = pltpu.async_copy(gb_hbm, gb_v, i_sem)

  def issue_gather(k):
    return pltpu.async_copy(word_hbm.at[idx_v.at[k]], word_bufs[k % NBUF],
                            g_sems[k % NBUF])

  def issue_pos(k):
    return pltpu.async_copy(pos_hbm.at[pl.ds(s0 + k * CHUNK, CHUNK)],
                            pos_bufs[k % 2], (pp0_sem, pp1_sem)[k % 2])

  pos_descs = {k: issue_pos(k) for k in range(2)}
  gather_descs = {}
  for k in range(NBUF - 1):
    for d in idx_descs[k * B:(k + 1) * B]:
      d.wait()
    gather_descs[k] = issue_gather(k)
  for d in idx_descs[(NBUF - 1) * B:]:
    d.wait()
  gb_desc.wait()
  out_descs = {}

  for k in range(NCHUNK):
    if k + NBUF - 1 < NCHUNK:
      if k - 1 >= 0:
        for d in out_descs[k - 1]:
          d.wait()
      gather_descs[k + NBUF - 1] = issue_gather(k + NBUF - 1)
    gather_descs[k].wait()
    pos_descs[k].wait()

    wbuf = word_bufs[k % NBUF]
    pbuf = pos_bufs[k % 2]

    @plsc.parallel_loop(0, NGROUPS, step=1, unroll=1)
    def gbody(g, wbuf=wbuf, pbuf=pbuf):
      gs = pl.ds(lax.mul(g, LANES), LANES)
      gms = [gb_v[b, gs] for b in range(B)]
      bts = [gb_v[B + b, gs] for b in range(B)]
      for r in range(CHUNK):
        p16 = pbuf[r, gs]
        for b in range(B):
          row = b * CHUNK + r
          wbuf[row, gs] = (wbuf[row, gs] + p16) * gms[b] + bts[b]

    if k + 2 < NCHUNK:
      pos_descs[k + 2] = issue_pos(k + 2)

    out_descs[k] = tuple(
        pltpu.async_copy(wbuf.at[pl.ds(b * CHUNK, CHUNK)],
                         out_hbm.at[b, pl.ds(s0 + k * CHUNK, CHUNK)],
                         o_sems[k % 2])
        for b in range(B))

  for k in range(max(0, NCHUNK - NBUF), NCHUNK):
    for d in out_descs[k]:
      d.wait()


def _sc_gather_affine(ids, word_emb, pos_emb, gb):
  kern = pl.kernel(
      _sc_body,
      out_type=jax.ShapeDtypeStruct((B, S, DIM), jnp.float32),
      mesh=plsc.VectorSubcoreMesh(core_axis_name="c", subcore_axis_name="s",
                                  num_cores=NC, num_subcores=NS),
      scratch_types=[
          pltpu.VMEM((B * CHUNK, DIM), jnp.float32),
          pltpu.VMEM((B * CHUNK, DIM), jnp.float32),
          pltpu.VMEM((B * CHUNK, DIM), jnp.float32),
          pltpu.VMEM((B * CHUNK, DIM), jnp.float32),
          pltpu.VMEM((CHUNK, DIM), jnp.float32),
          pltpu.VMEM((CHUNK, DIM), jnp.float32),
          pltpu.VMEM((NCHUNK, B * CHUNK), jnp.int32),
          pltpu.VMEM((2 * B, DIM), jnp.float32),
          pltpu.SemaphoreType.DMA,
          pltpu.SemaphoreType.DMA,
          pltpu.SemaphoreType.DMA,
          pltpu.SemaphoreType.DMA,
          pltpu.SemaphoreType.DMA,
          pltpu.SemaphoreType.DMA,
          pltpu.SemaphoreType.DMA,
          pltpu.SemaphoreType.DMA,
          pltpu.SemaphoreType.DMA,
      ],
  )
  return kern(ids, word_emb, pos_emb, gb)


def kernel(input_ids, condition_ids, word_emb, pos_emb, cond_emb, W_hidden,
           W_gma, W_bta, clngma, clnbta):
  if input_ids.dtype != jnp.int32:
    input_ids = input_ids.astype(jnp.int32)
  if condition_ids.dtype != jnp.int32:
    condition_ids = condition_ids.astype(jnp.int32)
  gb = _affine_params(condition_ids, cond_emb, W_hidden, W_gma, W_bta,
                      clngma, clnbta)
  return _sc_gather_affine(input_ids, word_emb, pos_emb, gb)


# speedup vs baseline: 1.0732x; 1.0035x over previous
import jax
import jax.numpy as jnp
from jax import lax
from jax.experimental import pallas as pl
from jax.experimental.pallas import tpu as pltpu
from jax.experimental.pallas import tpu_sc as plsc

B = 4
S = 2048
DIM = 768
COND_SIZE = 6
COND_DIMS = 128

NC = 2
NS = 16
NW = NC * NS
SROWS = S // NW
CHUNK = 8
NCHUNK = SROWS // CHUNK
NBUF = 4
LANES = 16
NGROUPS = DIM // LANES


KSTEPS = 2
JBLK = DIM // KSTEPS


def _affine_body(cid_ref, cond_emb_ref, wh_ref, wg_ref, wb_ref, clg_ref,
                 clb_ref, gb_ref, conds_ref, gt_ref, bt_ref):
  k = pl.program_id(0)

  @pl.when(k == 0)
  def _():
    conds_ref[...] = lax.dot_general(cond_emb_ref[...], wh_ref[...],
                                     (((1,), (1,)), ((), ())),
                                     preferred_element_type=jnp.float32)

  gt_ref[pl.ds(k * JBLK, JBLK), :] = lax.dot_general(
      wg_ref[...], conds_ref[...], (((1,), (1,)), ((), ())),
      preferred_element_type=jnp.float32)
  bt_ref[pl.ds(k * JBLK, JBLK), :] = lax.dot_general(
      wb_ref[...], conds_ref[...], (((1,), (1,)), ((), ())),
      preferred_element_type=jnp.float32)

  @pl.when(k == KSTEPS - 1)
  def _():
    onehot = (cid_ref[...][:, None]
              == lax.broadcasted_iota(jnp.int32, (B, COND_SIZE), 1)
              ).astype(jnp.float32)
    gb_ref[0:B, :] = clg_ref[...][None, :] + lax.dot_general(
        onehot, gt_ref[...], (((1,), (1,)), ((), ())),
        preferred_element_type=jnp.float32)
    gb_ref[B:2 * B, :] = clb_ref[...][None, :] + lax.dot_general(
        onehot, bt_ref[...], (((1,), (1,)), ((), ())),
        preferred_element_type=jnp.float32)


def _affine_params(condition_ids, cond_emb, W_hidden, W_gma, W_bta, clngma,
                   clnbta):
  return pl.pallas_call(
      _affine_body,
      grid=(KSTEPS,),
      in_specs=[
          pl.BlockSpec((B,), lambda k: (0,)),
          pl.BlockSpec((COND_SIZE, COND_DIMS), lambda k: (0, 0)),
          pl.BlockSpec((DIM, COND_DIMS), lambda k: (0, 0)),
          pl.BlockSpec((JBLK, DIM), lambda k: (k, 0)),
          pl.BlockSpec((JBLK, DIM), lambda k: (k, 0)),
          pl.BlockSpec((DIM,), lambda k: (0,)),
          pl.BlockSpec((DIM,), lambda k: (0,)),
      ],
      out_specs=pl.BlockSpec((2 * B, DIM), lambda k: (0, 0)),
      out_shape=jax.ShapeDtypeStruct((2 * B, DIM), jnp.float32),
      scratch_shapes=[
          pltpu.VMEM((COND_SIZE, DIM), jnp.float32),
          pltpu.VMEM((DIM, COND_SIZE), jnp.float32),
          pltpu.VMEM((DIM, COND_SIZE), jnp.float32),
      ],
  )(condition_ids, cond_emb, W_hidden, W_gma, W_bta, clngma, clnbta)


def _sc_body(ids_hbm, word_hbm, pos_hbm, gb_hbm, out_hbm,
             w0_v, w1_v, w2_v, w3_v, p0_v, p1_v, idx_v, gb_v,
             g0_sem, g1_sem, g2_sem, g3_sem, pp0_sem, pp1_sem, o0_sem,
             o1_sem, i_sem):
  wid = lax.axis_index("s") * NC + lax.axis_index("c")
  s0 = wid * SROWS

  word_bufs = (w0_v, w1_v, w2_v, w3_v)
  pos_bufs = (p0_v, p1_v)
  g_sems = (g0_sem, g1_sem, g2_sem, g3_sem)
  o_sems = (o0_sem, o1_sem)

  idx_descs = [
      pltpu.async_copy(ids_hbm.at[b, pl.ds(s0 + c * CHUNK, CHUNK)],
                       idx_v.at[c, pl.ds(b * CHUNK, CHUNK)],
                       g_sems[c % NBUF] if c < NBUF - 1 else i_sem)
      for c in range(NCHUNK) for b in range(B)
  ]
  gb_desc = pltpu.async_copy(gb_hbm, gb_v, i_sem)

  def issue_gather(k):
    return pltpu.async_copy(word_hbm.at[idx_v.at[k]], word_bufs[k % NBUF],
                            g_sems[k % NBUF])

  def issue_pos(k):
    return pltpu.async_copy(pos_hbm.at[pl.ds(s0 + k * CHUNK, CHUNK)],
                            pos_bufs[k % 2], (pp0_sem, pp1_sem)[k % 2])

  pos_descs = {k: issue_pos(k) for k in range(2)}
  gather_descs = {}
  for k in range(NBUF - 1):
    for d in idx_descs[k * B:(k + 1) * B]:
      d.wait()
    gather_descs[k] = issue_gather(k)
  for d in idx_descs[(NBUF - 1) * B:]:
    d.wait()
  gb_desc.wait()
  out_descs = {}

  for k in range(NCHUNK):
    if k + NBUF - 1 < NCHUNK:
      if k - 1 >= 0:
        for d in out_descs[k - 1]:
          d.wait()
      gather_descs[k + NBUF - 1] = issue_gather(k + NBUF - 1)
    gather_descs[k].wait()
    pos_descs[k].wait()

    wbuf = word_bufs[k % NBUF]
    pbuf = pos_bufs[k % 2]

    @plsc.parallel_loop(0, NGROUPS, step=1, unroll=1)
    def gbody(g, wbuf=wbuf, pbuf=pbuf):
      gs = pl.ds(lax.mul(g, LANES), LANES)
      gms = [gb_v[b, gs] for b in range(B)]
      bts = [gb_v[B + b, gs] for b in range(B)]
      for r in range(CHUNK):
        p16 = pbuf[r, gs]
        for b in range(B):
          row = b * CHUNK + r
          wbuf[row, gs] = (wbuf[row, gs] + p16) * gms[b] + bts[b]

    if k + 2 < NCHUNK:
      pos_descs[k + 2] = issue_pos(k + 2)

    out_descs[k] = tuple(
        pltpu.async_copy(wbuf.at[pl.ds(b * CHUNK, CHUNK)],
                         out_hbm.at[b, pl.ds(s0 + k * CHUNK, CHUNK)],
                         o_sems[k % 2])
        for b in range(B))

  for k in range(max(0, NCHUNK - NBUF), NCHUNK):
    for d in out_descs[k]:
      d.wait()


def _sc_gather_affine(ids, word_emb, pos_emb, gb):
  kern = pl.kernel(
      _sc_body,
      out_type=jax.ShapeDtypeStruct((B, S, DIM), jnp.float32),
      mesh=plsc.VectorSubcoreMesh(core_axis_name="c", subcore_axis_name="s",
                                  num_cores=NC, num_subcores=NS),
      scratch_types=[
          pltpu.VMEM((B * CHUNK, DIM), jnp.float32),
          pltpu.VMEM((B * CHUNK, DIM), jnp.float32),
          pltpu.VMEM((B * CHUNK, DIM), jnp.float32),
          pltpu.VMEM((B * CHUNK, DIM), jnp.float32),
          pltpu.VMEM((CHUNK, DIM), jnp.float32),
          pltpu.VMEM((CHUNK, DIM), jnp.float32),
          pltpu.VMEM((NCHUNK, B * CHUNK), jnp.int32),
          pltpu.VMEM((2 * B, DIM), jnp.float32),
          pltpu.SemaphoreType.DMA,
          pltpu.SemaphoreType.DMA,
          pltpu.SemaphoreType.DMA,
          pltpu.SemaphoreType.DMA,
          pltpu.SemaphoreType.DMA,
          pltpu.SemaphoreType.DMA,
          pltpu.SemaphoreType.DMA,
          pltpu.SemaphoreType.DMA,
          pltpu.SemaphoreType.DMA,
      ],
  )
  return kern(ids, word_emb, pos_emb, gb)


def kernel(input_ids, condition_ids, word_emb, pos_emb, cond_emb, W_hidden,
           W_gma, W_bta, clngma, clnbta):
  if input_ids.dtype != jnp.int32:
    input_ids = input_ids.astype(jnp.int32)
  if condition_ids.dtype != jnp.int32:
    condition_ids = condition_ids.astype(jnp.int32)
  gb = _affine_params(condition_ids, cond_emb, W_hidden, W_gma, W_bta,
                      clngma, clnbta)
  return _sc_gather_affine(input_ids, word_emb, pos_emb, gb)
